# baseline (speedup 1.0000x reference)
import jax
import jax.numpy as jnp
from jax.experimental import pallas as pl
from jax.experimental.pallas import tpu as pltpu

B, J = 16, 8
R, L = 96, 64 * 64


def _body(alpha_ref, maskT_ref, x_ref, o_ref):
    b = pl.program_id(0)
    j = pl.program_id(1)
    logits = maskT_ref[:] * alpha_ref[0, 0]
    m = jnp.max(logits, axis=0, keepdims=True)
    e = jnp.exp(logits - m)
    wT = e / jnp.sum(e, axis=0, keepdims=True)
    lane = jax.lax.broadcasted_iota(jnp.int32, (J, B), 1)
    row = jax.lax.broadcasted_iota(jnp.int32, (J, B), 0)
    w = jnp.sum(jnp.where((lane == b) & (row == j), wT, 0.0))

    x = x_ref[0, 0]

    @pl.when(j == 0)
    def _init():
        o_ref[0] = x * w

    @pl.when(j > 0)
    def _acc():
        o_ref[0] += x * w


def kernel(inps, mask, alpha):
    x = inps.reshape(B, J, R, L)
    maskT = mask.T
    alpha2 = jnp.reshape(alpha, (1, 1))
    out = pl.pallas_call(
        _body,
        grid=(B, J),
        in_specs=[
            pl.BlockSpec(memory_space=pltpu.SMEM),
            pl.BlockSpec((J, B), lambda b, j: (0, 0)),
            pl.BlockSpec((1, 1, R, L), lambda b, j: (b, j, 0, 0)),
        ],
        out_specs=pl.BlockSpec((1, R, L), lambda b, j: (b, 0, 0)),
        out_shape=jax.ShapeDtypeStruct((B, R, L), jnp.float32),
        compiler_params=pltpu.CompilerParams(
            dimension_semantics=("parallel", "arbitrary"),
        ),
    )(alpha2, maskT, x)
    sampled = out.reshape(B, 96, 64, 64)
    logp = jnp.zeros((B,), jnp.float32)
    return (sampled, logp)
